# baseline (speedup 1.0000x reference)
import functools

import jax
import jax.numpy as jnp
from jax import lax
from jax.experimental import pallas as pl
from jax.experimental.pallas import tpu as pltpu
from jax.experimental.pallas import tpu_sc as plsc

NT = 2048
NE = 8
ND = 768
NF = 3072
RB = 256
NB = (2 * NT + NE * (RB - 1) + RB - 1) // RB
PMAX = NB * RB
FB = 512
FT = NF // FB
NW = 32
CHUNK = NT // NW
LANES = 16
GW = 128


def _router_body(x_ref, wr_ref, d0_ref, d1_ref, g0_ref, g1_ref, bexp_ref):
    x = x_ref[...]
    wr = wr_ref[...]
    logits = jnp.dot(x, wr, preferred_element_type=jnp.float32)
    m = jnp.max(logits, axis=1, keepdims=True)
    ex = jnp.exp(logits - m)
    p = ex / jnp.sum(ex, axis=1, keepdims=True)

    lane = lax.broadcasted_iota(jnp.int32, (NT, NE), 1)
    p0 = jnp.max(p, axis=1, keepdims=True)
    i0 = jnp.min(jnp.where(p == p0, lane, NE), axis=1)
    pmask = jnp.where(lane == i0[:, None], -jnp.inf, p)
    p1 = jnp.max(pmask, axis=1, keepdims=True)
    i1 = jnp.min(jnp.where(pmask == p1, lane, NE), axis=1)
    s = p0 + p1
    g0_ref[...] = jnp.broadcast_to(p0 / s, (NT, GW))
    g1_ref[...] = jnp.broadcast_to(p1 / s, (NT, GW))

    oh0 = (lane == i0[:, None]).astype(jnp.bfloat16)
    oh1 = (lane == i1[:, None]).astype(jnp.bfloat16)
    r_i = lax.broadcasted_iota(jnp.int32, (NT, NT), 0)
    c_i = lax.broadcasted_iota(jnp.int32, (NT, NT), 1)
    tri = (c_i < r_i).astype(jnp.bfloat16)
    rank0 = jnp.dot(tri, oh0, preferred_element_type=jnp.float32)
    rank1 = jnp.dot(tri, oh1, preferred_element_type=jnp.float32)
    oh0f = oh0.astype(jnp.float32)
    oh1f = oh1.astype(jnp.float32)
    c0 = jnp.sum(oh0f, axis=0, keepdims=True)
    counts = c0 + jnp.sum(oh1f, axis=0, keepdims=True)
    rank1 = rank1 + c0

    padded = jnp.ceil(counts / RB) * RB
    eu = lax.broadcasted_iota(jnp.int32, (NE, NE), 0)
    ev = lax.broadcasted_iota(jnp.int32, (NE, NE), 1)
    incl = (eu <= ev).astype(jnp.float32)
    cum = jnp.dot(padded, incl, preferred_element_type=jnp.float32)
    base = cum - padded
    d0_ref[...] = jnp.sum(oh0f * (rank0 + base), axis=1).astype(jnp.int32)
    d1_ref[...] = jnp.sum(oh1f * (rank1 + base), axis=1).astype(jnp.int32)

    rows = lax.broadcasted_iota(jnp.int32, (128, NE), 0).astype(jnp.float32) * float(RB)
    bexp_ref[...] = jnp.sum((rows >= cum).astype(jnp.int32), axis=1)


def _router(x_flat, wr):
    return pl.pallas_call(
        _router_body,
        out_shape=(
            jax.ShapeDtypeStruct((NT,), jnp.int32),
            jax.ShapeDtypeStruct((NT,), jnp.int32),
            jax.ShapeDtypeStruct((NT, GW), jnp.float32),
            jax.ShapeDtypeStruct((NT, GW), jnp.float32),
            jax.ShapeDtypeStruct((128,), jnp.int32),
        ),
    )(x_flat, wr)


def _ffn_body(bexp_ref, xs_ref, gs_ref, w1_ref, b1_ref, w2_ref, b2_ref, out_ref):
    @pl.when(bexp_ref[pl.program_id(0)] < NE)
    def _run():
        acc = jnp.broadcast_to(b2_ref[0], (RB, ND))
        xb = xs_ref[...].astype(jnp.bfloat16)
        for t in range(FT):
            h = jnp.dot(xb, w1_ref[0, :, pl.ds(t * FB, FB)],
                        preferred_element_type=jnp.float32)
            h = jnp.maximum(h + b1_ref[0, :, pl.ds(t * FB, FB)], 0.0)
            acc = acc + jnp.dot(h.astype(jnp.bfloat16),
                                w2_ref[0, pl.ds(t * FB, FB), :],
                                preferred_element_type=jnp.float32)
        out_ref[...] = acc * gs_ref[:, :1]


def _ffn(bexp, xs, gs, w1, b1, w2, b2):
    def eix(b, bexp_ref):
        return jnp.minimum(bexp_ref[b], NE - 1)

    grid_spec = pltpu.PrefetchScalarGridSpec(
        num_scalar_prefetch=1,
        grid=(NB,),
        in_specs=[
            pl.BlockSpec((RB, ND), lambda b, bexp_ref: (b, 0)),
            pl.BlockSpec((RB, GW), lambda b, bexp_ref: (b, 0)),
            pl.BlockSpec((1, ND, NF), lambda b, bexp_ref: (eix(b, bexp_ref), 0, 0)),
            pl.BlockSpec((1, 1, NF), lambda b, bexp_ref: (eix(b, bexp_ref), 0, 0)),
            pl.BlockSpec((1, NF, ND), lambda b, bexp_ref: (eix(b, bexp_ref), 0, 0)),
            pl.BlockSpec((1, 1, ND), lambda b, bexp_ref: (eix(b, bexp_ref), 0, 0)),
        ],
        out_specs=pl.BlockSpec((RB, ND), lambda b, bexp_ref: (b, 0)),
    )
    return pl.pallas_call(
        _ffn_body,
        grid_spec=grid_spec,
        out_shape=jax.ShapeDtypeStruct((PMAX, ND), jnp.float32),
    )(bexp, xs, gs, w1.astype(jnp.bfloat16), b1.reshape(NE, 1, NF),
      w2.astype(jnp.bfloat16), b2.reshape(NE, 1, ND))


def _wid():
    return lax.axis_index("s") * 2 + lax.axis_index("c")


@functools.cache
def _sc_kernels():
    mesh = plsc.VectorSubcoreMesh(core_axis_name="c", subcore_axis_name="s")

    @functools.partial(
        pl.kernel,
        mesh=mesh,
        out_type=(
            jax.ShapeDtypeStruct((PMAX, ND), jnp.float32),
            jax.ShapeDtypeStruct((PMAX, GW), jnp.float32),
        ),
        scratch_types=[
            pltpu.VMEM((CHUNK,), jnp.int32),
            pltpu.VMEM((CHUNK, ND), jnp.float32),
            pltpu.VMEM((CHUNK, GW), jnp.float32),
            pltpu.SemaphoreType.DMA,
            pltpu.SemaphoreType.DMA,
        ],
    )
    def scatter_x(x_hbm, d0_hbm, d1_hbm, g0_hbm, g1_hbm,
                  xs_hbm, gs_hbm, idx_v, rows_v, gv_v, sem, semg):
        base = _wid() * CHUNK
        pltpu.sync_copy(x_hbm.at[pl.ds(base, CHUNK)], rows_v)
        pltpu.sync_copy(d0_hbm.at[pl.ds(base, CHUNK)], idx_v)
        pltpu.sync_copy(g0_hbm.at[pl.ds(base, CHUNK)], gv_v)
        cp = pltpu.async_copy(rows_v, xs_hbm.at[idx_v], sem)
        cpg = pltpu.async_copy(gv_v, gs_hbm.at[idx_v], semg)
        cp.wait()
        cpg.wait()
        pltpu.sync_copy(d1_hbm.at[pl.ds(base, CHUNK)], idx_v)
        pltpu.sync_copy(g1_hbm.at[pl.ds(base, CHUNK)], gv_v)
        cp = pltpu.async_copy(rows_v, xs_hbm.at[idx_v], sem)
        cpg = pltpu.async_copy(gv_v, gs_hbm.at[idx_v], semg)
        cp.wait()
        cpg.wait()

    @functools.partial(
        pl.kernel,
        mesh=mesh,
        out_type=jax.ShapeDtypeStruct((NT, ND), jnp.float32),
        scratch_types=[
            pltpu.VMEM((CHUNK,), jnp.int32),
            pltpu.VMEM((CHUNK,), jnp.int32),
            pltpu.VMEM((CHUNK, ND), jnp.float32),
            pltpu.VMEM((CHUNK, ND), jnp.float32),
            pltpu.SemaphoreType.DMA,
            pltpu.SemaphoreType.DMA,
        ],
    )
    def combine(ys_hbm, d0_hbm, d1_hbm, out_hbm,
                i0_v, i1_v, buf0, buf1, sem0, sem1):
        base = _wid() * CHUNK
        pltpu.sync_copy(d0_hbm.at[pl.ds(base, CHUNK)], i0_v)
        pltpu.sync_copy(d1_hbm.at[pl.ds(base, CHUNK)], i1_v)
        cp0 = pltpu.async_copy(ys_hbm.at[i0_v], buf0, sem0)
        cp1 = pltpu.async_copy(ys_hbm.at[i1_v], buf1, sem1)
        cp0.wait()
        cp1.wait()

        def body(i, carry):
            for j in range(ND // LANES):
                sl = pl.ds(j * LANES, LANES)
                buf0[i, sl] = buf0[i, sl] + buf1[i, sl]
            return carry

        lax.fori_loop(0, CHUNK, body, 0)
        pltpu.sync_copy(buf0, out_hbm.at[pl.ds(base, CHUNK)])

    return scatter_x, combine


def kernel(x, Wr, W1, b1, W2, b2):
    bb, tt, _ = x.shape
    x_flat = x.reshape(NT, ND)
    scatter_x, combine = _sc_kernels()
    d0, d1, g0, g1, bexp = _router(x_flat, Wr)
    xs, gs = scatter_x(x_flat, d0, d1, g0, g1)
    ys = _ffn(bexp, xs, gs, W1, b1, W2, b2)
    out = combine(ys, d0, d1)
    return out.reshape(bb, tt, ND)


# speedup vs baseline: 1.2629x; 1.2629x over previous
import functools

import jax
import jax.numpy as jnp
from jax import lax
from jax.experimental import pallas as pl
from jax.experimental.pallas import tpu as pltpu
from jax.experimental.pallas import tpu_sc as plsc

NT = 2048
NE = 8
ND = 768
NF = 3072
RB = 256
NB = (2 * NT + NE * (RB - 1) + RB - 1) // RB
PMAX = NB * RB
FB = 512
FT = NF // FB
NW = 32
CHUNK = NT // NW
LANES = 16
GW = 128


def _router_body(x_ref, wr_ref, d0_ref, d1_ref, g0_ref, g1_ref, bexp_ref):
    x = x_ref[...]
    wr = wr_ref[...]
    logits = jnp.dot(x, wr, preferred_element_type=jnp.float32)
    m = jnp.max(logits, axis=1, keepdims=True)
    ex = jnp.exp(logits - m)
    p = ex / jnp.sum(ex, axis=1, keepdims=True)

    lane = lax.broadcasted_iota(jnp.int32, (NT, NE), 1)
    p0 = jnp.max(p, axis=1, keepdims=True)
    i0 = jnp.min(jnp.where(p == p0, lane, NE), axis=1)
    pmask = jnp.where(lane == i0[:, None], -jnp.inf, p)
    p1 = jnp.max(pmask, axis=1, keepdims=True)
    i1 = jnp.min(jnp.where(pmask == p1, lane, NE), axis=1)
    s = p0 + p1
    g0_ref[...] = jnp.broadcast_to(p0 / s, (NT, GW))
    g1_ref[...] = jnp.broadcast_to(p1 / s, (NT, GW))

    oh0 = (lane == i0[:, None]).astype(jnp.bfloat16)
    oh1 = (lane == i1[:, None]).astype(jnp.bfloat16)
    r_i = lax.broadcasted_iota(jnp.int32, (NT, NT), 0)
    c_i = lax.broadcasted_iota(jnp.int32, (NT, NT), 1)
    tri = (c_i < r_i).astype(jnp.bfloat16)
    rank0 = jnp.dot(tri, oh0, preferred_element_type=jnp.float32)
    rank1 = jnp.dot(tri, oh1, preferred_element_type=jnp.float32)
    oh0f = oh0.astype(jnp.float32)
    oh1f = oh1.astype(jnp.float32)
    c0 = jnp.sum(oh0f, axis=0, keepdims=True)
    counts = c0 + jnp.sum(oh1f, axis=0, keepdims=True)
    rank1 = rank1 + c0

    padded = jnp.ceil(counts / RB) * RB
    eu = lax.broadcasted_iota(jnp.int32, (NE, NE), 0)
    ev = lax.broadcasted_iota(jnp.int32, (NE, NE), 1)
    incl = (eu <= ev).astype(jnp.float32)
    cum = jnp.dot(padded, incl, preferred_element_type=jnp.float32)
    base = cum - padded
    d0_ref[...] = jnp.sum(oh0f * (rank0 + base), axis=1).astype(jnp.int32)
    d1_ref[...] = jnp.sum(oh1f * (rank1 + base), axis=1).astype(jnp.int32)

    rows = lax.broadcasted_iota(jnp.int32, (128, NE), 0).astype(jnp.float32) * float(RB)
    bexp_ref[...] = jnp.sum((rows >= cum).astype(jnp.int32), axis=1)


def _router(x_flat, wr):
    return pl.pallas_call(
        _router_body,
        out_shape=(
            jax.ShapeDtypeStruct((NT,), jnp.int32),
            jax.ShapeDtypeStruct((NT,), jnp.int32),
            jax.ShapeDtypeStruct((NT, GW), jnp.float32),
            jax.ShapeDtypeStruct((NT, GW), jnp.float32),
            jax.ShapeDtypeStruct((128,), jnp.int32),
        ),
    )(x_flat, wr)


def _ffn_body(bexp_ref, xs_ref, gs_ref, w1_ref, b1_ref, w2_ref, b2_ref, out_ref):
    @pl.when(bexp_ref[pl.program_id(0)] < NE)
    def _run():
        acc = jnp.broadcast_to(b2_ref[0], (RB, ND))
        for t in range(FT):
            h = jnp.dot(xs_ref[...], w1_ref[0, :, pl.ds(t * FB, FB)],
                        preferred_element_type=jnp.float32)
            h = jnp.maximum(h + b1_ref[0, :, pl.ds(t * FB, FB)], 0.0)
            acc = acc + jnp.dot(h, w2_ref[0, pl.ds(t * FB, FB), :],
                                preferred_element_type=jnp.float32)
        out_ref[...] = acc * gs_ref[:, :1]


def _ffn(bexp, xs, gs, w1, b1, w2, b2):
    def eix(b, bexp_ref):
        return jnp.minimum(bexp_ref[b], NE - 1)

    grid_spec = pltpu.PrefetchScalarGridSpec(
        num_scalar_prefetch=1,
        grid=(NB,),
        in_specs=[
            pl.BlockSpec((RB, ND), lambda b, bexp_ref: (b, 0)),
            pl.BlockSpec((RB, GW), lambda b, bexp_ref: (b, 0)),
            pl.BlockSpec((1, ND, NF), lambda b, bexp_ref: (eix(b, bexp_ref), 0, 0)),
            pl.BlockSpec((1, 1, NF), lambda b, bexp_ref: (eix(b, bexp_ref), 0, 0)),
            pl.BlockSpec((1, NF, ND), lambda b, bexp_ref: (eix(b, bexp_ref), 0, 0)),
            pl.BlockSpec((1, 1, ND), lambda b, bexp_ref: (eix(b, bexp_ref), 0, 0)),
        ],
        out_specs=pl.BlockSpec((RB, ND), lambda b, bexp_ref: (b, 0)),
    )
    return pl.pallas_call(
        _ffn_body,
        grid_spec=grid_spec,
        out_shape=jax.ShapeDtypeStruct((PMAX, ND), jnp.float32),
    )(bexp, xs, gs, w1, b1.reshape(NE, 1, NF), w2, b2.reshape(NE, 1, ND))


def _wid():
    return lax.axis_index("s") * 2 + lax.axis_index("c")


@functools.cache
def _sc_kernels():
    mesh = plsc.VectorSubcoreMesh(core_axis_name="c", subcore_axis_name="s")

    @functools.partial(
        pl.kernel,
        mesh=mesh,
        out_type=(
            jax.ShapeDtypeStruct((PMAX, ND), jnp.float32),
            jax.ShapeDtypeStruct((PMAX, GW), jnp.float32),
        ),
        scratch_types=[
            pltpu.VMEM((CHUNK,), jnp.int32),
            pltpu.VMEM((CHUNK, ND), jnp.float32),
            pltpu.VMEM((CHUNK, GW), jnp.float32),
            pltpu.SemaphoreType.DMA,
            pltpu.SemaphoreType.DMA,
        ],
    )
    def scatter_x(x_hbm, d0_hbm, d1_hbm, g0_hbm, g1_hbm,
                  xs_hbm, gs_hbm, idx_v, rows_v, gv_v, sem, semg):
        base = _wid() * CHUNK
        pltpu.sync_copy(x_hbm.at[pl.ds(base, CHUNK)], rows_v)
        pltpu.sync_copy(d0_hbm.at[pl.ds(base, CHUNK)], idx_v)
        pltpu.sync_copy(g0_hbm.at[pl.ds(base, CHUNK)], gv_v)
        cp = pltpu.async_copy(rows_v, xs_hbm.at[idx_v], sem)
        cpg = pltpu.async_copy(gv_v, gs_hbm.at[idx_v], semg)
        cp.wait()
        cpg.wait()
        pltpu.sync_copy(d1_hbm.at[pl.ds(base, CHUNK)], idx_v)
        pltpu.sync_copy(g1_hbm.at[pl.ds(base, CHUNK)], gv_v)
        cp = pltpu.async_copy(rows_v, xs_hbm.at[idx_v], sem)
        cpg = pltpu.async_copy(gv_v, gs_hbm.at[idx_v], semg)
        cp.wait()
        cpg.wait()

    @functools.partial(
        pl.kernel,
        mesh=mesh,
        out_type=jax.ShapeDtypeStruct((NT, ND), jnp.float32),
        scratch_types=[
            pltpu.VMEM((CHUNK,), jnp.int32),
            pltpu.VMEM((CHUNK,), jnp.int32),
            pltpu.VMEM((CHUNK, ND), jnp.float32),
            pltpu.VMEM((CHUNK, ND), jnp.float32),
            pltpu.SemaphoreType.DMA,
            pltpu.SemaphoreType.DMA,
        ],
    )
    def combine(ys_hbm, d0_hbm, d1_hbm, out_hbm,
                i0_v, i1_v, buf0, buf1, sem0, sem1):
        base = _wid() * CHUNK
        pltpu.sync_copy(d0_hbm.at[pl.ds(base, CHUNK)], i0_v)
        pltpu.sync_copy(d1_hbm.at[pl.ds(base, CHUNK)], i1_v)
        cp0 = pltpu.async_copy(ys_hbm.at[i0_v], buf0, sem0)
        cp1 = pltpu.async_copy(ys_hbm.at[i1_v], buf1, sem1)
        cp0.wait()
        cp1.wait()

        def body(i, carry):
            for j in range(ND // LANES):
                sl = pl.ds(j * LANES, LANES)
                buf0[i, sl] = buf0[i, sl] + buf1[i, sl]
            return carry

        lax.fori_loop(0, CHUNK, body, 0)
        pltpu.sync_copy(buf0, out_hbm.at[pl.ds(base, CHUNK)])

    return scatter_x, combine


def kernel(x, Wr, W1, b1, W2, b2):
    bb, tt, _ = x.shape
    x_flat = x.reshape(NT, ND)
    scatter_x, combine = _sc_kernels()
    d0, d1, g0, g1, bexp = _router(x_flat, Wr)
    xs, gs = scatter_x(x_flat, d0, d1, g0, g1)
    ys = _ffn(bexp, xs, gs, W1, b1, W2, b2)
    out = combine(ys, d0, d1)
    return out.reshape(bb, tt, ND)
